# baseline (speedup 1.0000x reference)
import functools

import jax
import jax.numpy as jnp
from jax import lax
from jax.experimental import pallas as pl
from jax.experimental.pallas import tpu as pltpu
from jax.experimental.pallas import tpu_sc as plsc

_B = 16384
_D = 64
_NC = 2
_NS = 16
_NW = _NC * _NS
_BPW = _B // _NW
_CH = 128
_NCH = _BPW // _CH

_BLK = 1024


def _sc_gather(u_idx, i_idx, user_table, item_table):
    mesh = plsc.VectorSubcoreMesh(core_axis_name="c", subcore_axis_name="s")

    @functools.partial(
        pl.kernel,
        mesh=mesh,
        out_type=[
            jax.ShapeDtypeStruct((_B, _D), jnp.float32),
            jax.ShapeDtypeStruct((_B, _D), jnp.float32),
        ],
        scratch_types=[
            pltpu.VMEM((_NCH, _CH), jnp.int32),
            pltpu.VMEM((_NCH, _CH), jnp.int32),
            pltpu.VMEM((_BPW, _D), jnp.float32),
            pltpu.VMEM((_BPW, _D), jnp.float32),
            pltpu.SemaphoreType.DMA,
        ],
        compiler_params=pltpu.CompilerParams(use_tc_tiling_on_sc=False),
    )
    def gather_kernel(u_hbm, i_hbm, ut_hbm, it_hbm, ue_hbm, ie_hbm,
                      uidx_v, iidx_v, urows_v, irows_v, sem):
        wid = lax.axis_index("s") * _NC + lax.axis_index("c")
        base = wid * _BPW
        pltpu.sync_copy(u_hbm.at[wid], uidx_v)
        pltpu.sync_copy(i_hbm.at[wid], iidx_v)
        copies = []
        for j in range(_NCH):
            copies.append(pltpu.async_copy(
                ut_hbm.at[uidx_v.at[j]], urows_v.at[pl.ds(j * _CH, _CH)], sem))
            copies.append(pltpu.async_copy(
                it_hbm.at[iidx_v.at[j]], irows_v.at[pl.ds(j * _CH, _CH)], sem))
        for c in copies:
            c.wait()
        pltpu.sync_copy(urows_v, ue_hbm.at[pl.ds(base, _BPW)])
        pltpu.sync_copy(irows_v, ie_hbm.at[pl.ds(base, _BPW)])

    return gather_kernel(
        u_idx.reshape(_NW, _NCH, _CH),
        i_idx.reshape(_NW, _NCH, _CH),
        user_table,
        item_table,
    )


def _mlp_body(ue_ref, ie_ref, w1_ref, b1_ref, w2_ref, b2_ref,
              w3_ref, b3_ref, wp_ref, bp_ref, out_ref):
    dn = (((1,), (1,)), ((), ()))
    x = jnp.concatenate([ue_ref[...], ie_ref[...]], axis=1)
    h = lax.dot_general(x, w1_ref[...], dn, preferred_element_type=jnp.float32)
    h = jnp.maximum(h + b1_ref[...], 0.0)
    h = lax.dot_general(h, w2_ref[...], dn, preferred_element_type=jnp.float32)
    h = jnp.maximum(h + b2_ref[...], 0.0)
    h = lax.dot_general(h, w3_ref[...], dn, preferred_element_type=jnp.float32)
    h = jnp.maximum(h + b3_ref[...], 0.0)
    out_ref[...] = (
        jnp.sum(h * wp_ref[...], axis=1, keepdims=True) + bp_ref[0, 0])


def _tc_mlp(ue, ie, W1, b1, W2, b2, W3, b3, Wp, bp):
    grid = (_B // _BLK,)
    full = lambda shape: pl.BlockSpec(shape, lambda b: (0, 0))
    out = pl.pallas_call(
        _mlp_body,
        grid=grid,
        in_specs=[
            pl.BlockSpec((_BLK, _D), lambda b: (b, 0)),
            pl.BlockSpec((_BLK, _D), lambda b: (b, 0)),
            full(W1.shape), full((1, b1.shape[0])),
            full(W2.shape), full((1, b2.shape[0])),
            full(W3.shape), full((1, b3.shape[0])),
            full(Wp.shape), full((1, 1)),
        ],
        out_specs=pl.BlockSpec((_BLK, 1), lambda b: (b, 0)),
        out_shape=jax.ShapeDtypeStruct((_B, 1), jnp.float32),
    )(ue, ie, W1, b1[None, :], W2, b2[None, :], W3, b3[None, :], Wp,
      bp[None, :])
    return out[:, 0]


def kernel(u, i, user_table, item_table, W1, b1, W2, b2, W3, b3, Wp, bp):
    u32 = u.astype(jnp.int32)
    i32 = i.astype(jnp.int32)
    ue, ie = _sc_gather(u32, i32, user_table, item_table)
    return _tc_mlp(ue, ie, W1, b1, W2, b2, W3, b3, Wp, bp)


# speedup vs baseline: 1.5400x; 1.5400x over previous
import functools

import jax
import jax.numpy as jnp
from jax import lax
from jax.experimental import pallas as pl
from jax.experimental.pallas import tpu as pltpu
from jax.experimental.pallas import tpu_sc as plsc

_B = 16384
_V = 1000000
_D = 64
_NC = 2
_NS = 16
_NW = _NC * _NS
_BPW = _B // _NW

_TBLK = 4096
_BLK = 1024


def _transpose_body(src_ref, eye_ref, out_ref):
    out_ref[...] = lax.dot_general(
        src_ref[...], eye_ref[...], (((0,), (0,)), ((), ())),
        preferred_element_type=jnp.float32)


def _tc_transpose(src_t, eye):
    grid = (pl.cdiv(_V, _TBLK),)
    return pl.pallas_call(
        _transpose_body,
        grid=grid,
        in_specs=[
            pl.BlockSpec((_D, _TBLK), lambda b: (0, b)),
            pl.BlockSpec((_D, _D), lambda b: (0, 0)),
        ],
        out_specs=pl.BlockSpec((_TBLK, _D), lambda b: (b, 0)),
        out_shape=jax.ShapeDtypeStruct((_V, _D), jnp.float32),
    )(src_t, eye)


def _sc_gather(idx, table):
    mesh = plsc.VectorSubcoreMesh(core_axis_name="c", subcore_axis_name="s")

    @functools.partial(
        pl.kernel,
        mesh=mesh,
        out_type=jax.ShapeDtypeStruct((_B, _D), jnp.float32),
        scratch_types=[
            pltpu.VMEM((_BPW,), jnp.int32),
            pltpu.VMEM((_BPW, _D), jnp.float32),
            pltpu.SemaphoreType.DMA,
        ],
    )
    def gather_kernel(idx_hbm, tab_hbm, out_hbm, idx_v, rows_v, sem):
        wid = lax.axis_index("s") * _NC + lax.axis_index("c")
        base = pl.multiple_of(wid * _BPW, _BPW)
        pltpu.sync_copy(idx_hbm.at[pl.ds(base, _BPW)], idx_v)

        def grp(g, _):
            vec = idx_v[pl.ds(g * 16, 16)]
            for k in range(16):
                pltpu.async_copy(
                    tab_hbm.at[vec[k]], rows_v.at[g * 16 + k], sem)
            return _

        lax.fori_loop(0, _BPW // 16, grp, 0)
        pltpu.make_async_copy(
            tab_hbm.at[pl.ds(0, _BPW)], rows_v, sem).wait()
        pltpu.sync_copy(rows_v, out_hbm.at[pl.ds(base, _BPW)])

    return gather_kernel(idx, table)


def _mlp_body(ue_ref, ie_ref, w1_ref, b1_ref, w2_ref, b2_ref,
              w3_ref, b3_ref, wp_ref, bp_ref, out_ref):
    dn = (((1,), (1,)), ((), ()))
    x = jnp.concatenate([ue_ref[...], ie_ref[...]], axis=1)
    h = lax.dot_general(x, w1_ref[...], dn, preferred_element_type=jnp.float32)
    h = jnp.maximum(h + b1_ref[...], 0.0)
    h = lax.dot_general(h, w2_ref[...], dn, preferred_element_type=jnp.float32)
    h = jnp.maximum(h + b2_ref[...], 0.0)
    h = lax.dot_general(h, w3_ref[...], dn, preferred_element_type=jnp.float32)
    h = jnp.maximum(h + b3_ref[...], 0.0)
    out_ref[...] = (
        jnp.sum(h * wp_ref[...], axis=1, keepdims=True) + bp_ref[0, 0])


def _tc_mlp(ue, ie, W1, b1, W2, b2, W3, b3, Wp, bp):
    grid = (_B // _BLK,)
    full = lambda shape: pl.BlockSpec(shape, lambda b: (0, 0))
    out = pl.pallas_call(
        _mlp_body,
        grid=grid,
        in_specs=[
            pl.BlockSpec((_BLK, _D), lambda b: (b, 0)),
            pl.BlockSpec((_BLK, _D), lambda b: (b, 0)),
            full(W1.shape), full((1, b1.shape[0])),
            full(W2.shape), full((1, b2.shape[0])),
            full(W3.shape), full((1, b3.shape[0])),
            full(Wp.shape), full((1, 1)),
        ],
        out_specs=pl.BlockSpec((_BLK, 1), lambda b: (b, 0)),
        out_shape=jax.ShapeDtypeStruct((_B, 1), jnp.float32),
    )(ue, ie, W1, b1[None, :], W2, b2[None, :], W3, b3[None, :], Wp,
      bp[None, :])
    return out[:, 0]


def kernel(u, i, user_table, item_table, W1, b1, W2, b2, W3, b3, Wp, bp):
    u32 = u.astype(jnp.int32)
    i32 = i.astype(jnp.int32)
    eye = jnp.eye(_D, dtype=jnp.float32)
    ut = _tc_transpose(user_table.T, eye)
    ue = _sc_gather(u32, ut)
    it = _tc_transpose(item_table.T, eye)
    ie = _sc_gather(i32, it)
    return _tc_mlp(ue, ie, W1, b1, W2, b2, W3, b3, Wp, bp)


# speedup vs baseline: 1.9539x; 1.2688x over previous
import functools

import jax
import jax.numpy as jnp
from jax import lax
from jax.experimental import pallas as pl
from jax.experimental.pallas import tpu as pltpu
from jax.experimental.pallas import tpu_sc as plsc

_B = 16384
_V = 1000000
_S = 503808
_D = 64
_NC = 2
_NS = 16
_NW = _NC * _NS
_BPW = _B // _NW

_TBLK = 4096
_BLK = 1024


def _transpose_body(a_ref, b_ref, eye_ref, out_ref):
    dn = (((0,), (0,)), ((), ()))
    ta = lax.dot_general(a_ref[...], eye_ref[...], dn,
                         preferred_element_type=jnp.float32)
    tb = lax.dot_general(b_ref[...], eye_ref[...], dn,
                         preferred_element_type=jnp.float32)
    out_ref[...] = jnp.concatenate([ta, tb], axis=1)


def _tc_fold_transpose(src_t, eye):
    nblk = _S // _TBLK
    last = (_V - 1) // _TBLK
    return pl.pallas_call(
        _transpose_body,
        grid=(nblk,),
        in_specs=[
            pl.BlockSpec((_D, _TBLK), lambda b: (0, b)),
            pl.BlockSpec((_D, _TBLK),
                         lambda b: (0, jnp.minimum(b + nblk, last))),
            pl.BlockSpec((_D, _D), lambda b: (0, 0)),
        ],
        out_specs=pl.BlockSpec((_TBLK, 2 * _D), lambda b: (b, 0)),
        out_shape=jax.ShapeDtypeStruct((_S, 2 * _D), jnp.float32),
    )(src_t, src_t, eye)


def _sc_gather(idx, table):
    mesh = plsc.VectorSubcoreMesh(core_axis_name="c", subcore_axis_name="s")

    @functools.partial(
        pl.kernel,
        mesh=mesh,
        out_type=jax.ShapeDtypeStruct((_B, 2 * _D), jnp.float32),
        scratch_types=[
            pltpu.VMEM((_BPW,), jnp.int32),
            pltpu.VMEM((_BPW, 2 * _D), jnp.float32),
            pltpu.SemaphoreType.DMA,
        ],
    )
    def gather_kernel(idx_hbm, tab_hbm, out_hbm, idx_v, rows_v, sem):
        wid = lax.axis_index("s") * _NC + lax.axis_index("c")
        base = pl.multiple_of(wid * _BPW, _BPW)
        pltpu.sync_copy(idx_hbm.at[pl.ds(base, _BPW)], idx_v)

        def grp(g, _):
            vec = idx_v[pl.ds(g * 16, 16)]
            for k in range(16):
                pltpu.async_copy(
                    tab_hbm.at[vec[k]], rows_v.at[g * 16 + k], sem)
            return _

        lax.fori_loop(0, _BPW // 16, grp, 0)
        pltpu.make_async_copy(
            tab_hbm.at[pl.ds(0, _BPW)], rows_v, sem).wait()
        pltpu.sync_copy(rows_v, out_hbm.at[pl.ds(base, _BPW)])

    return gather_kernel(idx, table)


def _mlp_body(ue_ref, ie_ref, um_ref, im_ref, w1_ref, b1_ref, w2_ref, b2_ref,
              w3_ref, b3_ref, wp_ref, bp_ref, out_ref):
    um = um_ref[...]
    im = im_ref[...]
    ue = (1.0 - um) * ue_ref[:, :_D] + um * ue_ref[:, _D:]
    ie = (1.0 - im) * ie_ref[:, :_D] + im * ie_ref[:, _D:]
    dn = (((1,), (1,)), ((), ()))
    x = jnp.concatenate([ue, ie], axis=1)
    h = lax.dot_general(x, w1_ref[...], dn, preferred_element_type=jnp.float32)
    h = jnp.maximum(h + b1_ref[...], 0.0)
    h = lax.dot_general(h, w2_ref[...], dn, preferred_element_type=jnp.float32)
    h = jnp.maximum(h + b2_ref[...], 0.0)
    h = lax.dot_general(h, w3_ref[...], dn, preferred_element_type=jnp.float32)
    h = jnp.maximum(h + b3_ref[...], 0.0)
    out_ref[...] = (
        jnp.sum(h * wp_ref[...], axis=1, keepdims=True) + bp_ref[0, 0])


def _tc_mlp(ue, ie, um, im, W1, b1, W2, b2, W3, b3, Wp, bp):
    grid = (_B // _BLK,)
    full = lambda shape: pl.BlockSpec(shape, lambda b: (0, 0))
    out = pl.pallas_call(
        _mlp_body,
        grid=grid,
        in_specs=[
            pl.BlockSpec((_BLK, 2 * _D), lambda b: (b, 0)),
            pl.BlockSpec((_BLK, 2 * _D), lambda b: (b, 0)),
            pl.BlockSpec((_BLK, _D), lambda b: (b, 0)),
            pl.BlockSpec((_BLK, _D), lambda b: (b, 0)),
            full(W1.shape), full((1, b1.shape[0])),
            full(W2.shape), full((1, b2.shape[0])),
            full(W3.shape), full((1, b3.shape[0])),
            full(Wp.shape), full((1, 1)),
        ],
        out_specs=pl.BlockSpec((_BLK, 1), lambda b: (b, 0)),
        out_shape=jax.ShapeDtypeStruct((_B, 1), jnp.float32),
    )(ue, ie, um, im, W1, b1[None, :], W2, b2[None, :], W3, b3[None, :], Wp,
      bp[None, :])
    return out[:, 0]


def kernel(u, i, user_table, item_table, W1, b1, W2, b2, W3, b3, Wp, bp):
    u32 = u.astype(jnp.int32)
    i32 = i.astype(jnp.int32)
    um = jnp.broadcast_to((u32 >= _S).astype(jnp.float32)[:, None], (_B, _D))
    im = jnp.broadcast_to((i32 >= _S).astype(jnp.float32)[:, None], (_B, _D))
    umod = jnp.where(u32 < _S, u32, u32 - _S)
    imod = jnp.where(i32 < _S, i32, i32 - _S)
    eye = jnp.eye(_D, dtype=jnp.float32)
    ut = _tc_fold_transpose(user_table.T, eye)
    ue = _sc_gather(umod, ut)
    it = _tc_fold_transpose(item_table.T, eye)
    ie = _sc_gather(imod, it)
    return _tc_mlp(ue, ie, um, im, W1, b1, W2, b2, W3, b3, Wp, bp)


# speedup vs baseline: 2.2010x; 1.1265x over previous
import functools

import jax
import jax.numpy as jnp
from jax import lax
from jax.experimental import pallas as pl
from jax.experimental.pallas import tpu as pltpu
from jax.experimental.pallas import tpu_sc as plsc

_B = 16384
_V = 1000000
_S = 507904
_D = 64
_NC = 2
_NS = 16
_NW = _NC * _NS
_BPW = _B // _NW

_TBLK = 8192
_BLK = 1024


def _transpose_body(a_ref, b_ref, eye_ref, out_ref):
    dn = (((0,), (0,)), ((), ()))
    ta = lax.dot_general(a_ref[...], eye_ref[...], dn,
                         preferred_element_type=jnp.float32)
    tb = lax.dot_general(b_ref[...], eye_ref[...], dn,
                         preferred_element_type=jnp.float32)
    out_ref[...] = jnp.concatenate([ta, tb], axis=1)


def _tc_fold_transpose(src_t, eye):
    nblk = _S // _TBLK
    last = (_V - 1) // _TBLK
    return pl.pallas_call(
        _transpose_body,
        grid=(nblk,),
        in_specs=[
            pl.BlockSpec((_D, _TBLK), lambda b: (0, b)),
            pl.BlockSpec((_D, _TBLK),
                         lambda b: (0, jnp.minimum(b + nblk, last))),
            pl.BlockSpec((_D, _D), lambda b: (0, 0)),
        ],
        out_specs=pl.BlockSpec((_TBLK, 2 * _D), lambda b: (b, 0)),
        out_shape=jax.ShapeDtypeStruct((_S, 2 * _D), jnp.float32),
    )(src_t, src_t, eye)


def _sc_gather(idx, table):
    mesh = plsc.VectorSubcoreMesh(core_axis_name="c", subcore_axis_name="s")

    @functools.partial(
        pl.kernel,
        mesh=mesh,
        out_type=jax.ShapeDtypeStruct((_B, 2 * _D), jnp.float32),
        scratch_types=[
            pltpu.VMEM((_BPW,), jnp.int32),
            pltpu.VMEM((_BPW, 2 * _D), jnp.float32),
            pltpu.SemaphoreType.DMA,
        ],
    )
    def gather_kernel(idx_hbm, tab_hbm, out_hbm, idx_v, rows_v, sem):
        wid = lax.axis_index("s") * _NC + lax.axis_index("c")
        base = pl.multiple_of(wid * _BPW, _BPW)
        pltpu.sync_copy(idx_hbm.at[pl.ds(base, _BPW)], idx_v)

        def grp(g, _):
            vec = idx_v[pl.ds(g * 16, 16)]
            for k in range(16):
                pltpu.async_copy(
                    tab_hbm.at[vec[k]], rows_v.at[g * 16 + k], sem)
            return _

        lax.fori_loop(0, _BPW // 16, grp, 0)
        pltpu.make_async_copy(
            tab_hbm.at[pl.ds(0, _BPW)], rows_v, sem).wait()
        pltpu.sync_copy(rows_v, out_hbm.at[pl.ds(base, _BPW)])

    return gather_kernel(idx, table)


def _mlp_body(ue_ref, ie_ref, um_ref, im_ref, w1_ref, b1_ref, w2_ref, b2_ref,
              w3_ref, b3_ref, wp_ref, bp_ref, out_ref):
    um = um_ref[...]
    im = im_ref[...]
    ue = (1.0 - um) * ue_ref[:, :_D] + um * ue_ref[:, _D:]
    ie = (1.0 - im) * ie_ref[:, :_D] + im * ie_ref[:, _D:]
    dn = (((1,), (1,)), ((), ()))
    x = jnp.concatenate([ue, ie], axis=1)
    h = lax.dot_general(x, w1_ref[...], dn, preferred_element_type=jnp.float32)
    h = jnp.maximum(h + b1_ref[...], 0.0)
    h = lax.dot_general(h, w2_ref[...], dn, preferred_element_type=jnp.float32)
    h = jnp.maximum(h + b2_ref[...], 0.0)
    h = lax.dot_general(h, w3_ref[...], dn, preferred_element_type=jnp.float32)
    h = jnp.maximum(h + b3_ref[...], 0.0)
    out_ref[...] = (
        jnp.sum(h * wp_ref[...], axis=1, keepdims=True) + bp_ref[0, 0])


def _tc_mlp(ue, ie, um, im, W1, b1, W2, b2, W3, b3, Wp, bp):
    grid = (_B // _BLK,)
    full = lambda shape: pl.BlockSpec(shape, lambda b: (0, 0))
    out = pl.pallas_call(
        _mlp_body,
        grid=grid,
        in_specs=[
            pl.BlockSpec((_BLK, 2 * _D), lambda b: (b, 0)),
            pl.BlockSpec((_BLK, 2 * _D), lambda b: (b, 0)),
            pl.BlockSpec((_BLK, _D), lambda b: (b, 0)),
            pl.BlockSpec((_BLK, _D), lambda b: (b, 0)),
            full(W1.shape), full((1, b1.shape[0])),
            full(W2.shape), full((1, b2.shape[0])),
            full(W3.shape), full((1, b3.shape[0])),
            full(Wp.shape), full((1, 1)),
        ],
        out_specs=pl.BlockSpec((_BLK, 1), lambda b: (b, 0)),
        out_shape=jax.ShapeDtypeStruct((_B, 1), jnp.float32),
    )(ue, ie, um, im, W1, b1[None, :], W2, b2[None, :], W3, b3[None, :], Wp,
      bp[None, :])
    return out[:, 0]


def kernel(u, i, user_table, item_table, W1, b1, W2, b2, W3, b3, Wp, bp):
    u32 = u.astype(jnp.int32)
    i32 = i.astype(jnp.int32)
    um = jnp.broadcast_to((u32 >= _S).astype(jnp.float32)[:, None], (_B, _D))
    im = jnp.broadcast_to((i32 >= _S).astype(jnp.float32)[:, None], (_B, _D))
    umod = jnp.where(u32 < _S, u32, u32 - _S)
    imod = jnp.where(i32 < _S, i32, i32 - _S)
    eye = jnp.eye(_D, dtype=jnp.float32)
    ut = _tc_fold_transpose(user_table.T, eye)
    it = _tc_fold_transpose(item_table.T, eye)
    ue = _sc_gather(umod, ut)
    ie = _sc_gather(imod, it)
    return _tc_mlp(ue, ie, um, im, W1, b1, W2, b2, W3, b3, Wp, bp)


# speedup vs baseline: 2.3155x; 1.0520x over previous
import functools

import jax
import jax.numpy as jnp
from jax import lax
from jax.experimental import pallas as pl
from jax.experimental.pallas import tpu as pltpu
from jax.experimental.pallas import tpu_sc as plsc

_B = 16384
_V = 1000000
_S = 507904
_D = 64
_NC = 2
_NS = 16
_NW = _NC * _NS
_BPW = _B // _NW

_TBLK = 16384
_BLK = 1024


def _transpose_body(a_ref, b_ref, eye_ref, out_ref):
    dn = (((0,), (0,)), ((), ()))
    ta = lax.dot_general(a_ref[...], eye_ref[...], dn,
                         preferred_element_type=jnp.float32)
    tb = lax.dot_general(b_ref[...], eye_ref[...], dn,
                         preferred_element_type=jnp.float32)
    out_ref[...] = jnp.concatenate([ta, tb], axis=1)


def _tc_fold_transpose(src_t, eye):
    nblk = _S // _TBLK
    last = (_V - 1) // _TBLK
    return pl.pallas_call(
        _transpose_body,
        grid=(nblk,),
        in_specs=[
            pl.BlockSpec((_D, _TBLK), lambda b: (0, b)),
            pl.BlockSpec((_D, _TBLK),
                         lambda b: (0, jnp.minimum(b + nblk, last))),
            pl.BlockSpec((_D, _D), lambda b: (0, 0)),
        ],
        out_specs=pl.BlockSpec((_TBLK, 2 * _D), lambda b: (b, 0)),
        out_shape=jax.ShapeDtypeStruct((_S, 2 * _D), jnp.float32),
    )(src_t, src_t, eye)


def _sc_gather(idx, table):
    mesh = plsc.VectorSubcoreMesh(core_axis_name="c", subcore_axis_name="s")

    @functools.partial(
        pl.kernel,
        mesh=mesh,
        out_type=jax.ShapeDtypeStruct((_B, 2 * _D), jnp.float32),
        scratch_types=[
            pltpu.VMEM((_BPW,), jnp.int32),
            pltpu.VMEM((_BPW, 2 * _D), jnp.float32),
            pltpu.SemaphoreType.DMA,
        ],
    )
    def gather_kernel(idx_hbm, tab_hbm, out_hbm, idx_v, rows_v, sem):
        wid = lax.axis_index("s") * _NC + lax.axis_index("c")
        base = pl.multiple_of(wid * _BPW, _BPW)
        pltpu.sync_copy(idx_hbm.at[pl.ds(base, _BPW)], idx_v)

        def grp(g, _):
            vec = idx_v[pl.ds(g * 16, 16)]
            for k in range(16):
                pltpu.async_copy(
                    tab_hbm.at[vec[k]], rows_v.at[g * 16 + k], sem)
            return _

        lax.fori_loop(0, _BPW // 16, grp, 0)
        pltpu.make_async_copy(
            tab_hbm.at[pl.ds(0, _BPW)], rows_v, sem).wait()
        pltpu.sync_copy(rows_v, out_hbm.at[pl.ds(base, _BPW)])

    return gather_kernel(idx, table)


def _mlp_body(ue_ref, ie_ref, um_ref, im_ref, w1_ref, b1_ref, w2_ref, b2_ref,
              w3_ref, b3_ref, wp_ref, bp_ref, out_ref):
    um = um_ref[...]
    im = im_ref[...]
    ue = (1.0 - um) * ue_ref[:, :_D] + um * ue_ref[:, _D:]
    ie = (1.0 - im) * ie_ref[:, :_D] + im * ie_ref[:, _D:]
    dn = (((1,), (1,)), ((), ()))
    x = jnp.concatenate([ue, ie], axis=1)
    h = lax.dot_general(x, w1_ref[...], dn, preferred_element_type=jnp.float32)
    h = jnp.maximum(h + b1_ref[...], 0.0)
    h = lax.dot_general(h, w2_ref[...], dn, preferred_element_type=jnp.float32)
    h = jnp.maximum(h + b2_ref[...], 0.0)
    h = lax.dot_general(h, w3_ref[...], dn, preferred_element_type=jnp.float32)
    h = jnp.maximum(h + b3_ref[...], 0.0)
    out_ref[...] = (
        jnp.sum(h * wp_ref[...], axis=1, keepdims=True) + bp_ref[0, 0])


def _tc_mlp(ue, ie, um, im, W1, b1, W2, b2, W3, b3, Wp, bp):
    grid = (_B // _BLK,)
    full = lambda shape: pl.BlockSpec(shape, lambda b: (0, 0))
    out = pl.pallas_call(
        _mlp_body,
        grid=grid,
        in_specs=[
            pl.BlockSpec((_BLK, 2 * _D), lambda b: (b, 0)),
            pl.BlockSpec((_BLK, 2 * _D), lambda b: (b, 0)),
            pl.BlockSpec((_BLK, _D), lambda b: (b, 0)),
            pl.BlockSpec((_BLK, _D), lambda b: (b, 0)),
            full(W1.shape), full((1, b1.shape[0])),
            full(W2.shape), full((1, b2.shape[0])),
            full(W3.shape), full((1, b3.shape[0])),
            full(Wp.shape), full((1, 1)),
        ],
        out_specs=pl.BlockSpec((_BLK, 1), lambda b: (b, 0)),
        out_shape=jax.ShapeDtypeStruct((_B, 1), jnp.float32),
    )(ue, ie, um, im, W1, b1[None, :], W2, b2[None, :], W3, b3[None, :], Wp,
      bp[None, :])
    return out[:, 0]


def kernel(u, i, user_table, item_table, W1, b1, W2, b2, W3, b3, Wp, bp):
    u32 = u.astype(jnp.int32)
    i32 = i.astype(jnp.int32)
    um = jnp.broadcast_to((u32 >= _S).astype(jnp.float32)[:, None], (_B, _D))
    im = jnp.broadcast_to((i32 >= _S).astype(jnp.float32)[:, None], (_B, _D))
    umod = jnp.where(u32 < _S, u32, u32 - _S)
    imod = jnp.where(i32 < _S, i32, i32 - _S)
    eye = jnp.eye(_D, dtype=jnp.float32)
    ut = _tc_fold_transpose(user_table.T, eye)
    it = _tc_fold_transpose(item_table.T, eye)
    ue = _sc_gather(umod, ut)
    ie = _sc_gather(imod, it)
    return _tc_mlp(ue, ie, um, im, W1, b1, W2, b2, W3, b3, Wp, bp)
